# baseline (speedup 1.0000x reference)
import functools

import jax
import jax.numpy as jnp
from jax import lax
from jax.experimental import pallas as pl
from jax.experimental.pallas import tpu as pltpu
from jax.experimental.pallas import tpu_sc as plsc

N_ROWS = 1000
B = 1024
SEQ = 20
NW = 32
B_PER_W = B // NW
NBUF = 2


def _recip_body(x_ref, o_ref):
    x = x_ref[...]
    d = jnp.where(x == 0.0, jnp.float32(9999999.99), x)
    o_ref[...] = 1.0 / d


_recip_call = pl.pallas_call(
    _recip_body,
    out_shape=jax.ShapeDtypeStruct((N_ROWS, N_ROWS), jnp.float32),
)


_sc_mesh = plsc.VectorSubcoreMesh(core_axis_name="c", subcore_axis_name="s")


@functools.partial(
    pl.kernel,
    mesh=_sc_mesh,
    out_type=jax.ShapeDtypeStruct((B, SEQ, N_ROWS), jnp.float32),
    compiler_params=pltpu.CompilerParams(use_tc_tiling_on_sc=False),
    scratch_types=[
        pltpu.VMEM((B_PER_W, SEQ), jnp.int32),
        pltpu.VMEM((B_PER_W, SEQ), jnp.int32),
        pltpu.VMEM((NBUF, SEQ, N_ROWS), jnp.float32),
        pltpu.SemaphoreType.DMA,
        pltpu.SemaphoreType.DMA,
        pltpu.SemaphoreType.DMA,
        pltpu.SemaphoreType.DMA,
    ],
)
def _sc_gather(venue_hbm, poi_hbm, table_hbm, out_hbm,
               poi_v, idx_v, rows_v, sem_i, sem_g0, sem_g1, sem_o0):
    wid = lax.axis_index("s") * 2 + lax.axis_index("c")
    base_w = wid * B_PER_W
    sem_g = (sem_g0, sem_g1)

    pltpu.sync_copy(poi_hbm.at[pl.ds(base_w, B_PER_W)], poi_v)
    idx_copies = [
        pltpu.async_copy(venue_hbm.at[poi_v.at[j]], idx_v.at[j], sem_i)
        for j in range(B_PER_W)
    ]
    for c in idx_copies:
        c.wait()

    gathers = [None] * B_PER_W
    outs = [None] * B_PER_W
    for j in range(B_PER_W):
        p = j % NBUF
        if j >= NBUF:
            outs[j - NBUF].wait()
        gathers[j] = pltpu.async_copy(table_hbm.at[idx_v.at[j]],
                                      rows_v.at[p], sem_g[p])
        if j >= 1:
            gathers[j - 1].wait()
            outs[j - 1] = pltpu.async_copy(
                rows_v.at[(j - 1) % NBUF], out_hbm.at[base_w + j - 1], sem_o0)
    gathers[B_PER_W - 1].wait()
    outs[B_PER_W - 1] = pltpu.async_copy(
        rows_v.at[(B_PER_W - 1) % NBUF], out_hbm.at[base_w + B_PER_W - 1],
        sem_o0)
    outs[B_PER_W - 2].wait()
    outs[B_PER_W - 1].wait()


def kernel(venueid2coor, inputs_poi, poi_distance_matrix):
    recip = _recip_call(poi_distance_matrix)
    return _sc_gather(venueid2coor, inputs_poi, recip)


# speedup vs baseline: 1.0714x; 1.0714x over previous
import functools

import jax
import jax.numpy as jnp
from jax import lax
from jax.experimental import pallas as pl
from jax.experimental.pallas import tpu as pltpu
from jax.experimental.pallas import tpu_sc as plsc

N_ROWS = 1000
N_PAD = 1024
B = 1024
SEQ = 20
B_TOTAL = B * SEQ
NW = 32
B_PER_W = B_TOTAL // NW
L = 16
N_STEPS = B_PER_W // L
NBUF = 4


def _recip_body(x_ref, o_ref):
    x = x_ref[...]
    d = jnp.where(x == 0.0, jnp.float32(9999999.99), x)
    r = 1.0 / d
    o_ref[...] = jnp.concatenate(
        [r, jnp.full((N_ROWS, N_PAD - N_ROWS), 1.0, jnp.float32)], axis=1)


_recip_call = pl.pallas_call(
    _recip_body,
    out_shape=jax.ShapeDtypeStruct((N_ROWS, N_PAD), jnp.float32),
)


_sc_mesh = plsc.VectorSubcoreMesh(core_axis_name="c", subcore_axis_name="s")


@functools.partial(
    pl.kernel,
    mesh=_sc_mesh,
    out_type=jax.ShapeDtypeStruct((B_TOTAL, N_PAD), jnp.float32),
    compiler_params=pltpu.CompilerParams(
        use_tc_tiling_on_sc=True, needs_layout_passes=False),
    scratch_types=[
        pltpu.VMEM((N_ROWS,), jnp.int32),
        pltpu.VMEM((B_PER_W,), jnp.int32),
        pltpu.VMEM((NBUF, L, N_PAD), jnp.float32),
        pltpu.SemaphoreType.DMA,
        pltpu.SemaphoreType.DMA,
        pltpu.SemaphoreType.DMA,
        pltpu.SemaphoreType.DMA,
        pltpu.SemaphoreType.DMA,
    ],
)
def _sc_gather(venue_hbm, poi_hbm, table_hbm, out_hbm,
               venue_v, poi_v, rows_v, sem_o, *sem_g):
    wid = lax.axis_index("s") * 2 + lax.axis_index("c")
    base_w = wid * B_PER_W

    pltpu.sync_copy(venue_hbm, venue_v)
    pltpu.sync_copy(poi_hbm.at[pl.ds(base_w, B_PER_W)], poi_v)

    gathers = [None] * N_STEPS
    outs = [None] * N_STEPS
    for j in range(N_STEPS):
        p = j % NBUF
        if j >= NBUF:
            outs[j - NBUF].wait()
        v = poi_v[pl.ds(j * L, L)]
        idx = plsc.load_gather(venue_v, [v])
        gathers[j] = pltpu.async_copy(table_hbm.at[idx], rows_v.at[p],
                                      sem_g[p])
        if j >= 1:
            q = (j - 1) % NBUF
            gathers[j - 1].wait()
            outs[j - 1] = pltpu.async_copy(
                rows_v.at[q], out_hbm.at[pl.ds(base_w + (j - 1) * L, L)],
                sem_o)
    gathers[N_STEPS - 1].wait()
    outs[N_STEPS - 1] = pltpu.async_copy(
        rows_v.at[(N_STEPS - 1) % NBUF],
        out_hbm.at[pl.ds(base_w + (N_STEPS - 1) * L, L)], sem_o)
    for j in range(N_STEPS - NBUF, N_STEPS):
        if outs[j] is not None:
            outs[j].wait()


def kernel(venueid2coor, inputs_poi, poi_distance_matrix):
    recip = _recip_call(poi_distance_matrix)
    poi_flat = inputs_poi.reshape(-1)
    out = _sc_gather(venueid2coor, poi_flat, recip)
    return out[:, :N_ROWS].reshape(B, SEQ, N_ROWS)
